# baseline (speedup 1.0000x reference)
import functools

import jax
import jax.numpy as jnp
from jax import lax
from jax.experimental import pallas as pl
from jax.experimental.pallas import tpu as pltpu
from jax.experimental.pallas import tpu_sc as plsc


def _gather_body(num_cores, b_per_w, idx_hbm, table_hbm, out_hbm,
                 idx_v, rows_v, sem):
    wid = lax.axis_index("s") * num_cores + lax.axis_index("c")
    base = wid * b_per_w
    pltpu.sync_copy(idx_hbm.at[pl.ds(base, b_per_w)], idx_v)
    pltpu.async_copy(table_hbm.at[idx_v], rows_v, sem).wait()
    pltpu.sync_copy(rows_v, out_hbm.at[pl.ds(base, b_per_w)])


@functools.cache
def _build(B, V, D):
    info = plsc.get_sparse_core_info()
    nw = info.num_cores * info.num_subcores
    assert B % (8 * nw) == 0
    b_per_w = B // nw
    mesh = plsc.VectorSubcoreMesh(core_axis_name="c", subcore_axis_name="s")
    return pl.kernel(
        functools.partial(_gather_body, info.num_cores, b_per_w),
        mesh=mesh,
        out_type=jax.ShapeDtypeStruct((B, D), jnp.float32),
        scratch_types=[
            pltpu.VMEM((b_per_w,), jnp.int32),
            pltpu.VMEM((b_per_w, D), jnp.float32),
            pltpu.SemaphoreType.DMA,
        ],
        compiler_params=pltpu.CompilerParams(use_tc_tiling_on_sc=False),
    )


def kernel(context_ids, weight):
    B = context_ids.shape[0]
    V, D = weight.shape
    return _build(B, V, D)(context_ids.astype(jnp.int32), weight)


# speedup vs baseline: 3.8887x; 3.8887x over previous
import functools

import jax
import jax.numpy as jnp
from jax import lax
from jax.experimental import pallas as pl
from jax.experimental.pallas import tpu as pltpu
from jax.experimental.pallas import tpu_sc as plsc

_G = 16


def _gather_body(num_cores, b_per_w, D, idx_hbm, tab_t_hbm, out_t_hbm,
                 idx_v, idx_s, tile_v, buf_v, sem):
    wid = lax.axis_index("s") * num_cores + lax.axis_index("c")
    base = wid * b_per_w
    pltpu.sync_copy(idx_hbm.at[pl.ds(base, b_per_w)], idx_v)

    lane_iota = lax.iota(jnp.int32, _G)

    def group_body(g, _):
        i0 = g * _G
        col_vec = (idx_v[pl.ds(i0, _G)] >> 7) << 7
        for k in range(_G):
            col = pl.multiple_of(col_vec[k], 128)
            pltpu.async_copy(
                tab_t_hbm.at[:, pl.ds(col, 128)], tile_v.at[k], sem
            )
        for k in range(_G):
            pltpu.make_async_copy(
                tab_t_hbm.at[:, pl.ds(0, 128)], tile_v.at[k], sem
            ).wait()
        lvec = idx_v[pl.ds(i0, _G)] & 127
        for c in range(D):
            cvec = jnp.full((_G,), c, jnp.int32)
            vals = plsc.load_gather(tile_v, [lane_iota, cvec, lvec])
            buf_v[c, pl.ds(i0, _G)] = vals
        return ()

    lax.fori_loop(0, b_per_w // _G, group_body, (), unroll=False)

    pltpu.sync_copy(buf_v, out_t_hbm.at[:, pl.ds(base, b_per_w)])


@functools.cache
def _build(B, V, D):
    info = plsc.get_sparse_core_info()
    nw = info.num_cores * info.num_subcores
    assert B % (8 * nw) == 0
    b_per_w = B // nw
    mesh = plsc.VectorSubcoreMesh(core_axis_name="c", subcore_axis_name="s")
    return pl.kernel(
        functools.partial(_gather_body, info.num_cores, b_per_w, D),
        mesh=mesh,
        out_type=jax.ShapeDtypeStruct((D, B), jnp.float32),
        scratch_types=[
            pltpu.VMEM((b_per_w,), jnp.int32),
            pltpu.SMEM((b_per_w,), jnp.int32),
            pltpu.VMEM((_G, D, 128), jnp.float32),
            pltpu.VMEM((D, b_per_w), jnp.float32),
            pltpu.SemaphoreType.DMA,
        ],
        compiler_params=pltpu.CompilerParams(needs_layout_passes=False),
    )


def kernel(context_ids, weight):
    B = context_ids.shape[0]
    V, D = weight.shape
    out_t = _build(B, V, D)(context_ids.astype(jnp.int32), weight.T)
    return out_t.T
